# baseline (speedup 1.0000x reference)
import functools

import jax
import jax.numpy as jnp
from jax import lax
from jax.experimental import pallas as pl
from jax.experimental.pallas import tpu as pltpu
from jax.experimental.pallas import tpu_sc as plsc

P = 128
H = 128
NR = 4
U = 16
N_INST = 4096
BLK = 512
GRID = N_INST // BLK
LN2 = 0.6931471805599453

_NC = 2
_NS = 16


def _sc_body(orig_hbm, labels_hbm, d_hbm, uniq_hbm, inv_hbm,
             row_v, out_v, lbl_v, sv_v, uniq_v, inv_v):
    wid = lax.axis_index("s") * _NC + lax.axis_index("c")
    base = wid * 2
    pltpu.sync_copy(orig_hbm.at[pl.ds(base * P, 2 * P)], row_v)
    zeros = jnp.zeros((16,), jnp.int32)
    for i in range(16):
        out_v[pl.ds(i * 16, 16)] = zeros
    for k in range(2):
        for j in range(8):
            v16 = row_v[pl.ds(k * P + j * 16, 16)]
            cnt, last = plsc.scan_count(v16, mask=v16 < P)
            plsc.addupdate_scatter(out_v, [k * P + v16], cnt, mask=last)
    for k in range(2):
        j = base + k
        out_row = (j % NR) * U + j // NR
        pltpu.sync_copy(out_v.at[pl.ds(k * P, P)],
                        d_hbm.at[pl.ds(out_row * P, P)])

    @pl.when(wid == 0)
    def _unique():
        pltpu.sync_copy(labels_hbm, lbl_v)
        iota = lax.iota(jnp.int32, 16)
        skeys, spos = plsc.sort_key_val(lbl_v[...], iota)
        sv_v[...] = skeys
        prev = plsc.load_gather(sv_v, [jnp.maximum(iota - 1, 0)])
        first = (iota == 0) | (skeys != prev)
        rank = plsc.cumsum(first.astype(jnp.int32)) - 1
        plsc.store_scatter(inv_v, [spos], rank)
        uniq_v[...] = jnp.full((16,), P, jnp.int32)
        plsc.store_scatter(uniq_v, [rank], skeys)
        pltpu.sync_copy(uniq_v, uniq_hbm)
        pltpu.sync_copy(inv_v, inv_hbm)


_sc_routing = functools.partial(
    pl.kernel,
    out_type=[
        jax.ShapeDtypeStruct((NR * U * P,), jnp.int32),
        jax.ShapeDtypeStruct((U,), jnp.int32),
        jax.ShapeDtypeStruct((U,), jnp.int32),
    ],
    mesh=plsc.VectorSubcoreMesh(core_axis_name="c", subcore_axis_name="s"),
    scratch_types=[
        pltpu.VMEM((2 * P,), jnp.int32),
        pltpu.VMEM((2 * P,), jnp.int32),
        pltpu.VMEM((U,), jnp.int32),
        pltpu.VMEM((U,), jnp.int32),
        pltpu.VMEM((U,), jnp.int32),
        pltpu.VMEM((U,), jnp.int32),
    ],
    compiler_params=pltpu.CompilerParams(needs_layout_passes=False,
                                         skip_device_barrier=True),
)(_sc_body)


def _dg(a, b, ca, cb):
    return lax.dot_general(
        a, b, (((ca,), (cb,)), ((), ())),
        precision=lax.Precision.HIGHEST,
        preferred_element_type=jnp.float32,
    )


def _tc_body(x_ref, rel_ref, proto_ref, d_ref, uniq_ref, inv_ref,
             logits_ref, loss_ref, proto_out_ref, pn_ref):
    i = pl.program_id(0)

    @pl.when(i == 0)
    def _prologue():
        f32 = jnp.float32

        def fiota(shape, dim):
            return lax.broadcasted_iota(jnp.int32, shape, dim).astype(f32)

        eye = (lax.broadcasted_iota(jnp.int32, (U, U), 0) ==
               lax.broadcasted_iota(jnp.int32, (U, U), 1)).astype(f32)

        uniq_row = uniq_ref[...].astype(f32)
        inv_row = inv_ref[...].astype(f32)
        rank_col = _dg(eye, inv_row, 1, 1)
        valid_row = (uniq_row < float(P)).astype(f32)
        valid_col = _dg(eye, valid_row, 1, 1)
        n_valid = jnp.sum(valid_col)

        d64 = d_ref[...].astype(f32)
        d0 = d64[:, 0:1]
        s_nz = (d0 != float(P)).astype(f32)
        kv0 = d0[0:U] + d0[U:2 * U] + d0[2 * U:3 * U] + d0[3 * U:4 * U]
        az = (jnp.sum(kv0) == float(NR * U * P)).astype(f32)
        k_nz = (kv0 != float(NR * P)).astype(f32)
        k64 = jnp.concatenate([k_nz, k_nz, k_nz, k_nz], axis=0)
        dg64 = d64 * s_nz * k64

        e_mat = (rank_col == fiota((U, U), 1)).astype(f32)

        gc = (fiota((P, U), 0) ==
              jnp.minimum(uniq_row, float(P - 1))).astype(f32)
        proto = proto_ref[...]
        t_emb = _dg(gc, proto, 0, 0)

        loss_sum = 0.0
        n_eff = 0.0
        prop = jnp.zeros((U, H), f32)
        num_prop = jnp.zeros((U, 1), f32)
        for r in range(NR):
            proj_r = _dg(proto, rel_ref[r], 1, 0)
            d_r = dg64[r * U:(r + 1) * U]
            c_r = _dg(e_mat, d_r, 1, 0) * valid_col
            z_r = _dg(t_emb, proj_r, 1, 1)
            sim_r = 1.0 / (1.0 + jnp.exp(-z_r))
            loss_sum = loss_sum + jnp.sum(
                c_r * jnp.log(1.0 + jnp.exp(1.0 - 2.0 * sim_r)))
            n_eff = n_eff + jnp.sum(c_r)
            prop = prop + _dg(c_r, proj_r, 1, 0)
            num_prop = num_prop + jnp.sum(c_r, axis=1, keepdims=True)

        denom = jnp.where(num_prop > 0.0, num_prop, 1.0)
        upd = 0.5 * t_emb + 0.5 * prop / denom
        new_rows = jnp.where(num_prop > 0.0, upd, t_emb)
        delta = (new_rows - t_emb) * valid_col * (1.0 - az)
        sc_t = (fiota((P, U), 0) == uniq_row).astype(f32)
        proto_out = proto + _dg(sc_t, delta, 1, 0)
        proto_out_ref[...] = proto_out

        tot = n_valid * float(NR * P)
        loss = (1.0 - az) * ((loss_sum + (tot - n_eff) * LN2) / tot)
        loss_ref[...] = jnp.broadcast_to(loss, (1, 1))

        one_row = jnp.ones((1, P), f32)
        pn_ref[...] = _dg(one_row, proto_out * proto_out, 1, 1)

    x = x_ref[...]
    xn = jnp.sum(x * x, axis=1, keepdims=True)
    cross = _dg(x, proto_out_ref[...], 1, 1)
    logits_ref[...] = 2.0 * cross - xn - pn_ref[...]


@jax.jit
def _run(x, rel, proto, orig_flat, labels):
    d_flat, uniq, inv = _sc_routing(orig_flat, labels)
    return pl.pallas_call(
        _tc_body,
        grid=(GRID,),
        in_specs=[
            pl.BlockSpec((BLK, H), lambda i: (i, 0)),
            pl.BlockSpec((NR, H, H), lambda i: (0, 0, 0)),
            pl.BlockSpec((P, H), lambda i: (0, 0)),
            pl.BlockSpec((NR * U, P), lambda i: (0, 0)),
            pl.BlockSpec((1, U), lambda i: (0, 0)),
            pl.BlockSpec((1, U), lambda i: (0, 0)),
        ],
        out_specs=[
            pl.BlockSpec((BLK, P), lambda i: (i, 0)),
            pl.BlockSpec((1, 1), lambda i: (0, 0)),
            pl.BlockSpec((P, H), lambda i: (0, 0)),
        ],
        out_shape=[
            jax.ShapeDtypeStruct((N_INST, P), jnp.float32),
            jax.ShapeDtypeStruct((1, 1), jnp.float32),
            jax.ShapeDtypeStruct((P, H), jnp.float32),
        ],
        scratch_shapes=[pltpu.VMEM((1, P), jnp.float32)],
    )(x, rel, proto, d_flat.reshape(NR * U, P),
      uniq.reshape(1, U), inv.reshape(1, U))


def kernel(instance_embedding, relation_embedding, proto_table, rel_label_ids, labels):
    orig_flat = rel_label_ids.astype(jnp.int32).reshape(-1)
    logits, loss, proto_out = _run(
        instance_embedding, relation_embedding, proto_table, orig_flat,
        labels.astype(jnp.int32))
    return (logits, loss.reshape(()), proto_out)


# speedup vs baseline: 1.0258x; 1.0258x over previous
import functools

import jax
import jax.numpy as jnp
from jax import lax
from jax.experimental import pallas as pl
from jax.experimental.pallas import tpu as pltpu
from jax.experimental.pallas import tpu_sc as plsc

P = 128
H = 128
NR = 4
U = 16
N_INST = 4096
BLK = 512
GRID = N_INST // BLK
LN2 = 0.6931471805599453

_NC = 2
_NS = 16


_RPT = 4


def _sc_body(orig_hbm, labels_hbm, d_hbm, uniq_hbm, inv_hbm,
             row_v, out_v, lbl_v, sv_v, uniq_v, inv_v):
    wid = lax.axis_index("s")
    base = wid * _RPT
    pltpu.sync_copy(orig_hbm.at[pl.ds(base * P, _RPT * P)], row_v)
    zeros = jnp.zeros((16,), jnp.int32)
    for i in range(_RPT * 8):
        out_v[pl.ds(i * 16, 16)] = zeros
    for k in range(_RPT):
        for j in range(8):
            v16 = row_v[pl.ds(k * P + j * 16, 16)]
            cnt, last = plsc.scan_count(v16, mask=v16 < P)
            plsc.addupdate_scatter(out_v, [k * P + v16], cnt, mask=last)
    for k in range(_RPT):
        j = base + k
        out_row = (j % NR) * U + j // NR
        pltpu.sync_copy(out_v.at[pl.ds(k * P, P)],
                        d_hbm.at[pl.ds(out_row * P, P)])

    @pl.when(wid == 0)
    def _unique():
        pltpu.sync_copy(labels_hbm, lbl_v)
        iota = lax.iota(jnp.int32, 16)
        skeys, spos = plsc.sort_key_val(lbl_v[...], iota)
        sv_v[...] = skeys
        prev = plsc.load_gather(sv_v, [jnp.maximum(iota - 1, 0)])
        first = (iota == 0) | (skeys != prev)
        rank = plsc.cumsum(first.astype(jnp.int32)) - 1
        plsc.store_scatter(inv_v, [spos], rank)
        uniq_v[...] = jnp.full((16,), P, jnp.int32)
        plsc.store_scatter(uniq_v, [rank], skeys)
        pltpu.sync_copy(uniq_v, uniq_hbm)
        pltpu.sync_copy(inv_v, inv_hbm)


_sc_routing = functools.partial(
    pl.kernel,
    out_type=[
        jax.ShapeDtypeStruct((NR * U * P,), jnp.int32),
        jax.ShapeDtypeStruct((U,), jnp.int32),
        jax.ShapeDtypeStruct((U,), jnp.int32),
    ],
    mesh=plsc.VectorSubcoreMesh(core_axis_name="c", subcore_axis_name="s",
                                num_cores=1),
    scratch_types=[
        pltpu.VMEM((4 * P,), jnp.int32),
        pltpu.VMEM((4 * P,), jnp.int32),
        pltpu.VMEM((U,), jnp.int32),
        pltpu.VMEM((U,), jnp.int32),
        pltpu.VMEM((U,), jnp.int32),
        pltpu.VMEM((U,), jnp.int32),
    ],
    compiler_params=pltpu.CompilerParams(needs_layout_passes=False,
                                         skip_device_barrier=True),
)(_sc_body)


def _dg(a, b, ca, cb):
    return lax.dot_general(
        a, b, (((ca,), (cb,)), ((), ())),
        precision=lax.Precision.HIGHEST,
        preferred_element_type=jnp.float32,
    )


def _tc_body(x_ref, rel_ref, proto_ref, d_ref, uniq_ref, inv_ref,
             logits_ref, loss_ref, proto_out_ref, pn_ref):
    i = pl.program_id(0)

    @pl.when(i == 0)
    def _prologue():
        f32 = jnp.float32

        def fiota(shape, dim):
            return lax.broadcasted_iota(jnp.int32, shape, dim).astype(f32)

        eye = (lax.broadcasted_iota(jnp.int32, (U, U), 0) ==
               lax.broadcasted_iota(jnp.int32, (U, U), 1)).astype(f32)

        uniq_row = uniq_ref[...].astype(f32)
        inv_row = inv_ref[...].astype(f32)
        rank_col = _dg(eye, inv_row, 1, 1)
        valid_row = (uniq_row < float(P)).astype(f32)
        valid_col = _dg(eye, valid_row, 1, 1)
        n_valid = jnp.sum(valid_col)

        d64 = d_ref[...].astype(f32)
        d0 = d64[:, 0:1]
        s_nz = (d0 != float(P)).astype(f32)
        kv0 = d0[0:U] + d0[U:2 * U] + d0[2 * U:3 * U] + d0[3 * U:4 * U]
        az = (jnp.sum(kv0) == float(NR * U * P)).astype(f32)
        k_nz = (kv0 != float(NR * P)).astype(f32)
        k64 = jnp.concatenate([k_nz, k_nz, k_nz, k_nz], axis=0)
        dg64 = d64 * s_nz * k64

        e_mat = (rank_col == fiota((U, U), 1)).astype(f32)

        gc = (fiota((P, U), 0) ==
              jnp.minimum(uniq_row, float(P - 1))).astype(f32)
        proto = proto_ref[...]
        t_emb = _dg(gc, proto, 0, 0)

        loss_sum = 0.0
        n_eff = 0.0
        prop = jnp.zeros((U, H), f32)
        num_prop = jnp.zeros((U, 1), f32)
        for r in range(NR):
            proj_r = _dg(proto, rel_ref[r], 1, 0)
            d_r = dg64[r * U:(r + 1) * U]
            c_r = _dg(e_mat, d_r, 1, 0) * valid_col
            z_r = _dg(t_emb, proj_r, 1, 1)
            sim_r = 1.0 / (1.0 + jnp.exp(-z_r))
            loss_sum = loss_sum + jnp.sum(
                c_r * jnp.log(1.0 + jnp.exp(1.0 - 2.0 * sim_r)))
            n_eff = n_eff + jnp.sum(c_r)
            prop = prop + _dg(c_r, proj_r, 1, 0)
            num_prop = num_prop + jnp.sum(c_r, axis=1, keepdims=True)

        denom = jnp.where(num_prop > 0.0, num_prop, 1.0)
        upd = 0.5 * t_emb + 0.5 * prop / denom
        new_rows = jnp.where(num_prop > 0.0, upd, t_emb)
        delta = (new_rows - t_emb) * valid_col * (1.0 - az)
        sc_t = (fiota((P, U), 0) == uniq_row).astype(f32)
        proto_out = proto + _dg(sc_t, delta, 1, 0)
        proto_out_ref[...] = proto_out

        tot = n_valid * float(NR * P)
        loss = (1.0 - az) * ((loss_sum + (tot - n_eff) * LN2) / tot)
        loss_ref[...] = jnp.broadcast_to(loss, (1, 1))

        one_row = jnp.ones((1, P), f32)
        pn_ref[...] = _dg(one_row, proto_out * proto_out, 1, 1)

    x = x_ref[...]
    xn = jnp.sum(x * x, axis=1, keepdims=True)
    cross = _dg(x, proto_out_ref[...], 1, 1)
    logits_ref[...] = 2.0 * cross - xn - pn_ref[...]


@jax.jit
def _run(x, rel, proto, orig_flat, labels):
    d_flat, uniq, inv = _sc_routing(orig_flat, labels)
    return pl.pallas_call(
        _tc_body,
        grid=(GRID,),
        in_specs=[
            pl.BlockSpec((BLK, H), lambda i: (i, 0)),
            pl.BlockSpec((NR, H, H), lambda i: (0, 0, 0)),
            pl.BlockSpec((P, H), lambda i: (0, 0)),
            pl.BlockSpec((NR * U, P), lambda i: (0, 0)),
            pl.BlockSpec((1, U), lambda i: (0, 0)),
            pl.BlockSpec((1, U), lambda i: (0, 0)),
        ],
        out_specs=[
            pl.BlockSpec((BLK, P), lambda i: (i, 0)),
            pl.BlockSpec((1, 1), lambda i: (0, 0)),
            pl.BlockSpec((P, H), lambda i: (0, 0)),
        ],
        out_shape=[
            jax.ShapeDtypeStruct((N_INST, P), jnp.float32),
            jax.ShapeDtypeStruct((1, 1), jnp.float32),
            jax.ShapeDtypeStruct((P, H), jnp.float32),
        ],
        scratch_shapes=[pltpu.VMEM((1, P), jnp.float32)],
    )(x, rel, proto, d_flat.reshape(NR * U, P),
      uniq.reshape(1, U), inv.reshape(1, U))


def kernel(instance_embedding, relation_embedding, proto_table, rel_label_ids, labels):
    orig_flat = rel_label_ids.astype(jnp.int32).reshape(-1)
    logits, loss, proto_out = _run(
        instance_embedding, relation_embedding, proto_table, orig_flat,
        labels.astype(jnp.int32))
    return (logits, loss.reshape(()), proto_out)
